# baseline (speedup 1.0000x reference)
import functools

import jax
import jax.numpy as jnp
from jax import lax
from jax.experimental import pallas as pl
from jax.experimental.pallas import tpu as pltpu
from jax.experimental.pallas import tpu_sc as plsc

N_ENT = 10000
NP = 10240
N_REL = 500
D = 128
E = 320000
B = 1024

NC = 2
NS = 16
DCH = 80
DPT = (E // (NC * NS)) // DCH
CH = 80
NG = 8
RING = 4
NGRP = 32
CPT = NG * NGRP
EP = NS * CPT * CH
ROWS_PER_TILE = NP // NS
_MESH = plsc.VectorSubcoreMesh(core_axis_name="c", subcore_axis_name="s")


@functools.partial(
    pl.kernel,
    out_type=(
        jax.ShapeDtypeStruct((NC, NP), jnp.float32),
        jax.ShapeDtypeStruct((B, D), jnp.float32),
        jax.ShapeDtypeStruct((B, D), jnp.float32),
    ),
    mesh=_MESH,
    scratch_types=[
        pltpu.VMEM_SHARED((NP,), jnp.float32),
        pltpu.VMEM((DCH,), jnp.float32),
        pltpu.VMEM((DPT, DCH), jnp.int32),
        pltpu.VMEM((B // (NC * NS),), jnp.int32),
        pltpu.VMEM((B // (NC * NS), D), jnp.float32),
        pltpu.VMEM((B // (NC * NS), D), jnp.float32),
        pltpu.SemaphoreType.DMA,
        pltpu.SemaphoreType.DMA,
        pltpu.SemaphoreType.DMA,
    ],
)
def _sc_deg_rel(dst3_hbm, rel_hbm, relr_hbm, reli_hbm, ones_hbm, zeros1_hbm,
                deg_out, rr_out, ri_out,
                deg_acc, ones_v, didx_v, ridx_v, rrows_v, irows_v,
                sem_r, sem_i, sem_s):
    c = lax.axis_index("c")
    s = lax.axis_index("s")
    wid = s * NC + c
    bpt = B // (NC * NS)

    pltpu.sync_copy(rel_hbm.at[pl.ds(wid * bpt, bpt)], ridx_v)
    hr = pltpu.async_copy(relr_hbm.at[ridx_v], rrows_v, sem_r)
    hi = pltpu.async_copy(reli_hbm.at[ridx_v], irows_v, sem_i)

    pltpu.sync_copy(zeros1_hbm, deg_acc.at[pl.ds(s * ROWS_PER_TILE, ROWS_PER_TILE)])
    pltpu.sync_copy(ones_hbm, ones_v)
    pltpu.sync_copy(dst3_hbm.at[wid], didx_v)
    hr.wait()
    pltpu.sync_copy(rrows_v, rr_out.at[pl.ds(wid * bpt, bpt)])
    hi.wait()
    pltpu.sync_copy(irows_v, ri_out.at[pl.ds(wid * bpt, bpt)])
    plsc.subcore_barrier()

    LAG = 8

    def body(i, carry):
        @pl.when(i >= LAG)
        def _():
            pltpu.make_async_copy(ones_v, deg_acc.at[didx_v.at[0]], sem_s).wait()
        pltpu.async_copy(ones_v, deg_acc.at[didx_v.at[i]], sem_s, add=True)
        return carry

    lax.fori_loop(0, DPT, body, 0)
    for _ in range(LAG):
        pltpu.make_async_copy(ones_v, deg_acc.at[didx_v.at[0]], sem_s).wait()
    plsc.subcore_barrier()
    pltpu.sync_copy(deg_acc.at[pl.ds(s * ROWS_PER_TILE, ROWS_PER_TILE)],
                    deg_out.at[c, pl.ds(s * ROWS_PER_TILE, ROWS_PER_TILE)])


@functools.partial(
    pl.kernel,
    out_type=(
        jax.ShapeDtypeStruct((NP, D), jnp.float32),
        jax.ShapeDtypeStruct((NP, D), jnp.float32),
    ),
    mesh=_MESH,
    scratch_types=[
        pltpu.VMEM_SHARED((NP, D), jnp.float32),
        pltpu.VMEM((NG, CH), jnp.int32),
        pltpu.VMEM((NG, CH), jnp.int32),
        pltpu.VMEM((NG, CH), jnp.int32),
        pltpu.VMEM((NG, CH), jnp.int32),
        [pltpu.VMEM((CH, D), jnp.float32)] * RING,
        pltpu.SemaphoreType.DMA,
        pltpu.SemaphoreType.DMA,
        [pltpu.SemaphoreType.DMA] * RING,
        [pltpu.SemaphoreType.DMA] * RING,
    ],
)
def _sc_scatter(hr_hbm, hi_hbm, src4_hbm, dst4_hbm, zeros2_hbm,
                sr_out, si_out,
                acc, sb0, sb1, db0, db1, bufs,
                isem0, isem1, gsems, ssems):
    c = lax.axis_index("c")
    s = lax.axis_index("s")
    sbanks, dbanks = (sb0, sb1), (db0, db1)
    isems = (isem0, isem1)

    pltpu.sync_copy(zeros2_hbm, acc.at[pl.ds(s * ROWS_PER_TILE, ROWS_PER_TILE)])
    plsc.subcore_barrier()

    def run(table):
        def wait_g(b):
            pltpu.make_async_copy(table.at[sb0.at[0]], bufs[b], gsems[b]).wait()

        def wait_s(b):
            pltpu.make_async_copy(bufs[b], acc.at[db0.at[0]], ssems[b]).wait()

        def wait_i(bank):
            pltpu.make_async_copy(src4_hbm.at[0, 0], sbanks[bank],
                                  isems[bank]).wait()
            pltpu.make_async_copy(dst4_hbm.at[0, 0], dbanks[bank],
                                  isems[bank]).wait()

        pltpu.sync_copy(src4_hbm.at[s, 0], sb0)
        pltpu.sync_copy(dst4_hbm.at[s, 0], db0)
        pltpu.async_copy(src4_hbm.at[s, 1], sb1, isem1)
        pltpu.async_copy(dst4_hbm.at[s, 1], db1, isem1)
        pltpu.async_copy(table.at[sb0.at[0]], bufs[0], gsems[0])
        pltpu.async_copy(table.at[sb0.at[1]], bufs[1], gsems[1])

        def outer(gp, carry):
            for gg in (0, 1):
                g = gp * 2 + gg
                sb, db = sbanks[gg], dbanks[gg]
                sbn = sbanks[1 - gg]
                for k in range(NG):
                    j = g * NG + k
                    b = k % RING
                    bn = (k + 2) % RING
                    wait_g(b)
                    pltpu.async_copy(bufs[b], acc.at[db.at[k]], ssems[b],
                                     add=True)
                    if k == 2:
                        g1 = g + 1
                        if gg == 0:
                            @pl.when(gp >= 1)
                            def _(g1=g1):
                                pltpu.async_copy(src4_hbm.at[s, g1], sb1,
                                                 isems[1])
                                pltpu.async_copy(dst4_hbm.at[s, g1], db1,
                                                 isems[1])
                        else:
                            @pl.when(gp < NGRP // 2 - 1)
                            def _(g1=g1):
                                pltpu.async_copy(src4_hbm.at[s, g1], sb0,
                                                 isems[0])
                                pltpu.async_copy(dst4_hbm.at[s, g1], db0,
                                                 isems[0])
                    @pl.when(j >= 2)
                    def _(bn=bn):
                        wait_s(bn)
                    if k == NG - 2:
                        if gg == 0:
                            wait_i(1)
                        else:
                            @pl.when(gp < NGRP // 2 - 1)
                            def _():
                                wait_i(0)
                    if k < NG - 2:
                        pltpu.async_copy(table.at[sb.at[k + 2]], bufs[bn],
                                         gsems[bn])
                    else:
                        @pl.when(j + 2 < CPT)
                        def _(k=k, bn=bn, sbn=sbn):
                            pltpu.async_copy(table.at[sbn.at[k - (NG - 2)]],
                                             bufs[bn], gsems[bn])
            return carry

        lax.fori_loop(0, NGRP // 2, outer, 0)
        wait_s(2)
        wait_s(3)

    @pl.when(c == 0)
    def _():
        run(hr_hbm)

    @pl.when(c == 1)
    def _():
        run(hi_hbm)

    plsc.subcore_barrier()
    sl = pl.ds(s * ROWS_PER_TILE, ROWS_PER_TILE)

    @pl.when(c == 0)
    def _():
        pltpu.sync_copy(acc.at[sl], sr_out.at[sl])

    @pl.when(c == 1)
    def _():
        pltpu.sync_copy(acc.at[sl], si_out.at[sl])


@functools.partial(
    pl.kernel,
    out_type=(
        jax.ShapeDtypeStruct((B, D), jnp.float32),
        jax.ShapeDtypeStruct((B, D), jnp.float32),
    ),
    mesh=_MESH,
    scratch_types=[
        pltpu.VMEM((B // (NC * NS),), jnp.int32),
        pltpu.VMEM((B // (NC * NS), D), jnp.float32),
        pltpu.SemaphoreType.DMA,
    ],
)
def _sc_gather_e1(hr_hbm, hi_hbm, e1_hbm, gr_out, gi_out, idx_v, rows_v, sem):
    c = lax.axis_index("c")
    s = lax.axis_index("s")
    wid = s * NC + c
    bpt = B // (NC * NS)
    sl = pl.ds(wid * bpt, bpt)
    pltpu.sync_copy(e1_hbm.at[sl], idx_v)
    pltpu.async_copy(hr_hbm.at[idx_v], rows_v, sem).wait()
    pltpu.sync_copy(rows_v, gr_out.at[sl])
    pltpu.async_copy(hi_hbm.at[idx_v], rows_v, sem).wait()
    pltpu.sync_copy(rows_v, gi_out.at[sl])


_BM = 1024


def _prep_body(degt_ref, er_ref, ei_ref, d_ref, hr_ref, hi_ref):
    deg = degt_ref[...]
    tot = deg[:, 0:1] + deg[:, 1:2]
    d = lax.rsqrt(jnp.maximum(tot, 1.0))
    d_ref[...] = d
    hr_ref[...] = er_ref[...] * d
    hi_ref[...] = ei_ref[...] * d


def _tc_prep(degt, er, ei):
    return pl.pallas_call(
        _prep_body,
        grid=(NP // _BM,),
        in_specs=[
            pl.BlockSpec((_BM, 2), lambda i: (i, 0)),
            pl.BlockSpec((_BM, D), lambda i: (i, 0)),
            pl.BlockSpec((_BM, D), lambda i: (i, 0)),
        ],
        out_specs=[
            pl.BlockSpec((_BM, 1), lambda i: (i, 0)),
            pl.BlockSpec((_BM, D), lambda i: (i, 0)),
            pl.BlockSpec((_BM, D), lambda i: (i, 0)),
        ],
        out_shape=[
            jax.ShapeDtypeStruct((NP, 1), jnp.float32),
            jax.ShapeDtypeStruct((NP, D), jnp.float32),
            jax.ShapeDtypeStruct((NP, D), jnp.float32),
        ],
    )(degt, er, ei)


def _layer_body(do_relu, post_scale, out_dtype, s_ref, d_ref, w_ref, o_ref):
    d = d_ref[...]
    x = s_ref[...].astype(jnp.float32) * d
    y = jnp.dot(x, w_ref[...], preferred_element_type=jnp.float32)
    if do_relu:
        y = jnp.maximum(y, 0.0)
    if post_scale:
        y = y * d
    o_ref[...] = y.astype(out_dtype)


def _tc_layer(s, dvec, w, do_relu, post_scale, out_dtype):
    return pl.pallas_call(
        functools.partial(_layer_body, do_relu, post_scale, out_dtype),
        grid=(NP // _BM,),
        in_specs=[
            pl.BlockSpec((_BM, D), lambda i: (i, 0)),
            pl.BlockSpec((_BM, 1), lambda i: (i, 0)),
            pl.BlockSpec((D, D), lambda i: (0, 0)),
        ],
        out_specs=pl.BlockSpec((_BM, D), lambda i: (i, 0)),
        out_shape=jax.ShapeDtypeStruct((NP, D), out_dtype),
    )(s, dvec, w)


_BN = 2048


def _score_body(ar_ref, ai_ref, br_ref, bi_ref, er_ref, ei_ref, o_ref):
    ar, ai = ar_ref[...], ai_ref[...]
    br, bi = br_ref[...], bi_ref[...]
    qr = ar * br - ai * bi
    qi = ar * bi + ai * br
    dn = (((1,), (1,)), ((), ()))
    sc = lax.dot_general(qr, er_ref[...], dn, preferred_element_type=jnp.float32)
    sc = sc + lax.dot_general(qi, ei_ref[...], dn,
                              preferred_element_type=jnp.float32)
    o_ref[...] = 1.0 / (1.0 + jnp.exp(-sc))


def _tc_score(ar, ai, br, bi, er, ei):
    return pl.pallas_call(
        _score_body,
        grid=(NP // _BN,),
        in_specs=[
            pl.BlockSpec((B, D), lambda j: (0, 0)),
            pl.BlockSpec((B, D), lambda j: (0, 0)),
            pl.BlockSpec((B, D), lambda j: (0, 0)),
            pl.BlockSpec((B, D), lambda j: (0, 0)),
            pl.BlockSpec((_BN, D), lambda j: (j, 0)),
            pl.BlockSpec((_BN, D), lambda j: (j, 0)),
        ],
        out_specs=pl.BlockSpec((B, _BN), lambda j: (0, j)),
        out_shape=jax.ShapeDtypeStruct((B, NP), jnp.float32),
    )(ar, ai, br, bi, er, ei)


def kernel(e1, rel, edge_index, emb_e_real, emb_e_img, emb_rel_real,
           emb_rel_img, W1, W2):
    f32 = jnp.float32
    e1 = e1.astype(jnp.int32)
    rel = rel.astype(jnp.int32)
    src = edge_index[0].astype(jnp.int32)
    dst = edge_index[1].astype(jnp.int32)
    full = jnp.full((EP // CH, CH), NP - 1, jnp.int32)
    src4 = lax.dynamic_update_slice(full, src.reshape(E // CH, CH),
                                    (0, 0)).reshape(NS, NGRP, NG, CH)
    dst4 = lax.dynamic_update_slice(full, dst.reshape(E // CH, CH),
                                    (0, 0)).reshape(NS, NGRP, NG, CH)
    dst3d = dst.reshape(NC * NS, DPT, DCH)

    er = emb_e_real.astype(f32)
    ei = emb_e_img.astype(f32)
    ones_c = jnp.ones((DCH,), f32)
    zeros1 = jnp.zeros((ROWS_PER_TILE,), f32)
    zeros2 = jnp.zeros((ROWS_PER_TILE, D), f32)

    deg_parts, r_r, r_i = _sc_deg_rel(dst3d, rel, emb_rel_real.astype(f32),
                                      emb_rel_img.astype(f32), ones_c, zeros1)
    dvec, h0r, h0i = _tc_prep(deg_parts.T, er, ei)

    s1r, s1i = _sc_scatter(h0r, h0i, src4, dst4, zeros2)
    h1r = _tc_layer(s1r, dvec, W1, do_relu=True, post_scale=True,
                    out_dtype=jnp.float32)
    h1i = _tc_layer(s1i, dvec, W1, do_relu=True, post_scale=True,
                    out_dtype=jnp.float32)

    s2r, s2i = _sc_scatter(h1r, h1i, src4, dst4, zeros2)
    h2r = _tc_layer(s2r, dvec, W2, do_relu=False, post_scale=False,
                    out_dtype=jnp.float32)
    h2i = _tc_layer(s2i, dvec, W2, do_relu=False, post_scale=False,
                    out_dtype=jnp.float32)

    g_r, g_i = _sc_gather_e1(h2r, h2i, e1)
    logits = _tc_score(g_r, g_i, r_r, r_i, er, ei)
    return logits[:, :N_ENT]


# speedup vs baseline: 1.0691x; 1.0691x over previous
import functools

import jax
import jax.numpy as jnp
from jax import lax
from jax.experimental import pallas as pl
from jax.experimental.pallas import tpu as pltpu
from jax.experimental.pallas import tpu_sc as plsc

N_ENT = 10000
NP = 10240
N_REL = 500
D = 128
E = 320000
B = 1024

NC = 2
NS = 16
DCH = 80
DPT = (E // (NC * NS)) // DCH
CH = 80
NG = 8
RING = 4
NGRP = 32
CPT = NG * NGRP
EP = NS * CPT * CH
ROWS_PER_TILE = NP // NS
_MESH = plsc.VectorSubcoreMesh(core_axis_name="c", subcore_axis_name="s")


@functools.partial(
    pl.kernel,
    out_type=(
        jax.ShapeDtypeStruct((NC, NP), jnp.float32),
        jax.ShapeDtypeStruct((B, D), jnp.float32),
        jax.ShapeDtypeStruct((B, D), jnp.float32),
    ),
    mesh=_MESH,
    scratch_types=[
        pltpu.VMEM_SHARED((NP,), jnp.float32),
        pltpu.VMEM((DCH,), jnp.float32),
        pltpu.VMEM((DPT, DCH), jnp.int32),
        pltpu.VMEM((B // (NC * NS),), jnp.int32),
        pltpu.VMEM((B // (NC * NS), D), jnp.float32),
        pltpu.VMEM((B // (NC * NS), D), jnp.float32),
        pltpu.SemaphoreType.DMA,
        pltpu.SemaphoreType.DMA,
        pltpu.SemaphoreType.DMA,
    ],
)
def _sc_deg_rel(dst3_hbm, rel_hbm, relr_hbm, reli_hbm, ones_hbm, zeros1_hbm,
                deg_out, rr_out, ri_out,
                deg_acc, ones_v, didx_v, ridx_v, rrows_v, irows_v,
                sem_r, sem_i, sem_s):
    c = lax.axis_index("c")
    s = lax.axis_index("s")
    wid = s * NC + c
    bpt = B // (NC * NS)

    pltpu.sync_copy(rel_hbm.at[pl.ds(wid * bpt, bpt)], ridx_v)
    hr = pltpu.async_copy(relr_hbm.at[ridx_v], rrows_v, sem_r)
    hi = pltpu.async_copy(reli_hbm.at[ridx_v], irows_v, sem_i)

    pltpu.sync_copy(zeros1_hbm, deg_acc.at[pl.ds(s * ROWS_PER_TILE, ROWS_PER_TILE)])
    pltpu.sync_copy(ones_hbm, ones_v)
    pltpu.sync_copy(dst3_hbm.at[wid], didx_v)
    hr.wait()
    pltpu.sync_copy(rrows_v, rr_out.at[pl.ds(wid * bpt, bpt)])
    hi.wait()
    pltpu.sync_copy(irows_v, ri_out.at[pl.ds(wid * bpt, bpt)])
    plsc.subcore_barrier()

    LAG = 8

    def body(i, carry):
        @pl.when(i >= LAG)
        def _():
            pltpu.make_async_copy(ones_v, deg_acc.at[didx_v.at[0]], sem_s).wait()
        pltpu.async_copy(ones_v, deg_acc.at[didx_v.at[i]], sem_s, add=True)
        return carry

    lax.fori_loop(0, DPT, body, 0)
    for _ in range(LAG):
        pltpu.make_async_copy(ones_v, deg_acc.at[didx_v.at[0]], sem_s).wait()
    plsc.subcore_barrier()
    pltpu.sync_copy(deg_acc.at[pl.ds(s * ROWS_PER_TILE, ROWS_PER_TILE)],
                    deg_out.at[c, pl.ds(s * ROWS_PER_TILE, ROWS_PER_TILE)])


@functools.partial(
    pl.kernel,
    out_type=(
        jax.ShapeDtypeStruct((NP, D), jnp.float32),
        jax.ShapeDtypeStruct((NP, D), jnp.float32),
    ),
    mesh=_MESH,
    scratch_types=[
        pltpu.VMEM_SHARED((NP, D), jnp.float32),
        pltpu.VMEM((NG, CH), jnp.int32),
        pltpu.VMEM((NG, CH), jnp.int32),
        pltpu.VMEM((NG, CH), jnp.int32),
        pltpu.VMEM((NG, CH), jnp.int32),
        [pltpu.VMEM((CH, D), jnp.float32)] * RING,
        pltpu.SemaphoreType.DMA,
        pltpu.SemaphoreType.DMA,
        [pltpu.SemaphoreType.DMA] * RING,
        [pltpu.SemaphoreType.DMA] * RING,
    ],
)
def _sc_scatter(hr_hbm, hi_hbm, src4_hbm, dst4_hbm, zeros2_hbm,
                sr_out, si_out,
                acc, sb0, sb1, db0, db1, bufs,
                isem0, isem1, gsems, ssems):
    c = lax.axis_index("c")
    s = lax.axis_index("s")
    sbanks, dbanks = (sb0, sb1), (db0, db1)
    isems = (isem0, isem1)

    pltpu.sync_copy(zeros2_hbm, acc.at[pl.ds(s * ROWS_PER_TILE, ROWS_PER_TILE)])
    plsc.subcore_barrier()

    def run(table):
        def wait_g(b):
            pltpu.make_async_copy(table.at[sb0.at[0]], bufs[b], gsems[b]).wait()

        def wait_s(b):
            pltpu.make_async_copy(bufs[b], acc.at[db0.at[0]], ssems[b]).wait()

        def wait_i(bank):
            pltpu.make_async_copy(src4_hbm.at[0, 0], sbanks[bank],
                                  isems[bank]).wait()
            pltpu.make_async_copy(dst4_hbm.at[0, 0], dbanks[bank],
                                  isems[bank]).wait()

        pltpu.sync_copy(src4_hbm.at[s, 0], sb0)
        pltpu.sync_copy(dst4_hbm.at[s, 0], db0)
        pltpu.async_copy(src4_hbm.at[s, 1], sb1, isem1)
        pltpu.async_copy(dst4_hbm.at[s, 1], db1, isem1)
        pltpu.async_copy(table.at[sb0.at[0]], bufs[0], gsems[0])
        pltpu.async_copy(table.at[sb0.at[1]], bufs[1], gsems[1])

        def outer(gp, carry):
            for gg in (0, 1):
                g = gp * 2 + gg
                sb, db = sbanks[gg], dbanks[gg]
                sbn = sbanks[1 - gg]
                for k in range(NG):
                    j = g * NG + k
                    b = k % RING
                    bn = (k + 2) % RING
                    wait_g(b)
                    pltpu.async_copy(bufs[b], acc.at[db.at[k]], ssems[b],
                                     add=True)
                    if k == 2:
                        g1 = g + 1
                        if gg == 0:
                            @pl.when(gp >= 1)
                            def _(g1=g1):
                                pltpu.async_copy(src4_hbm.at[s, g1], sb1,
                                                 isems[1])
                                pltpu.async_copy(dst4_hbm.at[s, g1], db1,
                                                 isems[1])
                        else:
                            @pl.when(gp < NGRP // 2 - 1)
                            def _(g1=g1):
                                pltpu.async_copy(src4_hbm.at[s, g1], sb0,
                                                 isems[0])
                                pltpu.async_copy(dst4_hbm.at[s, g1], db0,
                                                 isems[0])
                    @pl.when(j >= 2)
                    def _(bn=bn):
                        wait_s(bn)
                    if k == NG - 2:
                        if gg == 0:
                            wait_i(1)
                        else:
                            @pl.when(gp < NGRP // 2 - 1)
                            def _():
                                wait_i(0)
                    if k < NG - 2:
                        pltpu.async_copy(table.at[sb.at[k + 2]], bufs[bn],
                                         gsems[bn])
                    else:
                        @pl.when(j + 2 < CPT)
                        def _(k=k, bn=bn, sbn=sbn):
                            pltpu.async_copy(table.at[sbn.at[k - (NG - 2)]],
                                             bufs[bn], gsems[bn])
            return carry

        lax.fori_loop(0, NGRP // 2, outer, 0)
        wait_s(2)
        wait_s(3)

    @pl.when(c == 0)
    def _():
        run(hr_hbm)

    @pl.when(c == 1)
    def _():
        run(hi_hbm)

    plsc.subcore_barrier()
    sl = pl.ds(s * ROWS_PER_TILE, ROWS_PER_TILE)

    @pl.when(c == 0)
    def _():
        pltpu.sync_copy(acc.at[sl], sr_out.at[sl])

    @pl.when(c == 1)
    def _():
        pltpu.sync_copy(acc.at[sl], si_out.at[sl])


@functools.partial(
    pl.kernel,
    out_type=(
        jax.ShapeDtypeStruct((B, D), jnp.float32),
        jax.ShapeDtypeStruct((B, D), jnp.float32),
    ),
    mesh=_MESH,
    scratch_types=[
        pltpu.VMEM((B // (NC * NS),), jnp.int32),
        pltpu.VMEM((B // (NC * NS), D), jnp.float32),
        pltpu.SemaphoreType.DMA,
    ],
)
def _sc_gather_e1(hr_hbm, hi_hbm, e1_hbm, gr_out, gi_out, idx_v, rows_v, sem):
    c = lax.axis_index("c")
    s = lax.axis_index("s")
    wid = s * NC + c
    bpt = B // (NC * NS)
    sl = pl.ds(wid * bpt, bpt)
    pltpu.sync_copy(e1_hbm.at[sl], idx_v)
    pltpu.async_copy(hr_hbm.at[idx_v], rows_v, sem).wait()
    pltpu.sync_copy(rows_v, gr_out.at[sl])
    pltpu.async_copy(hi_hbm.at[idx_v], rows_v, sem).wait()
    pltpu.sync_copy(rows_v, gi_out.at[sl])


_BM = 1024


def _prep_body(degt_ref, er_ref, ei_ref, d_ref, hr_ref, hi_ref):
    deg = degt_ref[...]
    tot = deg[:, 0:1] + deg[:, 1:2]
    d = lax.rsqrt(jnp.maximum(tot, 1.0))
    d_ref[...] = d
    hr_ref[...] = er_ref[...] * d
    hi_ref[...] = ei_ref[...] * d


def _tc_prep(degt, er, ei):
    return pl.pallas_call(
        _prep_body,
        grid=(NP // _BM,),
        in_specs=[
            pl.BlockSpec((_BM, 2), lambda i: (i, 0)),
            pl.BlockSpec((_BM, D), lambda i: (i, 0)),
            pl.BlockSpec((_BM, D), lambda i: (i, 0)),
        ],
        out_specs=[
            pl.BlockSpec((_BM, 1), lambda i: (i, 0)),
            pl.BlockSpec((_BM, D), lambda i: (i, 0)),
            pl.BlockSpec((_BM, D), lambda i: (i, 0)),
        ],
        out_shape=[
            jax.ShapeDtypeStruct((NP, 1), jnp.float32),
            jax.ShapeDtypeStruct((NP, D), jnp.float32),
            jax.ShapeDtypeStruct((NP, D), jnp.float32),
        ],
    )(degt, er, ei)


def _layer_body(do_relu, post_scale, out_dtype, s_ref, d_ref, w_ref, o_ref):
    d = d_ref[...]
    x = s_ref[...].astype(jnp.float32) * d
    y = jnp.dot(x, w_ref[...], preferred_element_type=jnp.float32)
    if do_relu:
        y = jnp.maximum(y, 0.0)
    if post_scale:
        y = y * d
    o_ref[...] = y.astype(out_dtype)


def _tc_layer(s, dvec, w, do_relu, post_scale, out_dtype):
    return pl.pallas_call(
        functools.partial(_layer_body, do_relu, post_scale, out_dtype),
        grid=(NP // _BM,),
        in_specs=[
            pl.BlockSpec((_BM, D), lambda i: (i, 0)),
            pl.BlockSpec((_BM, 1), lambda i: (i, 0)),
            pl.BlockSpec((D, D), lambda i: (0, 0)),
        ],
        out_specs=pl.BlockSpec((_BM, D), lambda i: (i, 0)),
        out_shape=jax.ShapeDtypeStruct((NP, D), out_dtype),
    )(s, dvec, w)


_BN = 2048


def _score_body(ar_ref, ai_ref, br_ref, bi_ref, er_ref, ei_ref, o_ref):
    ar, ai = ar_ref[...], ai_ref[...]
    br, bi = br_ref[...], bi_ref[...]
    qr = ar * br - ai * bi
    qi = ar * bi + ai * br
    dn = (((1,), (1,)), ((), ()))
    sc = lax.dot_general(qr, er_ref[...], dn, preferred_element_type=jnp.float32)
    sc = sc + lax.dot_general(qi, ei_ref[...], dn, preferred_element_type=jnp.float32)
    o_ref[...] = 1.0 / (1.0 + jnp.exp(-sc))


def _tc_score(ar, ai, br, bi, er, ei):
    return pl.pallas_call(
        _score_body,
        grid=(NP // _BN,),
        in_specs=[
            pl.BlockSpec((B, D), lambda j: (0, 0)),
            pl.BlockSpec((B, D), lambda j: (0, 0)),
            pl.BlockSpec((B, D), lambda j: (0, 0)),
            pl.BlockSpec((B, D), lambda j: (0, 0)),
            pl.BlockSpec((_BN, D), lambda j: (j, 0)),
            pl.BlockSpec((_BN, D), lambda j: (j, 0)),
        ],
        out_specs=pl.BlockSpec((B, _BN), lambda j: (0, j)),
        out_shape=jax.ShapeDtypeStruct((B, NP), jnp.float32),
    )(ar, ai, br, bi, er, ei)


def kernel(e1, rel, edge_index, emb_e_real, emb_e_img, emb_rel_real,
           emb_rel_img, W1, W2):
    f32 = jnp.float32
    e1 = e1.astype(jnp.int32)
    rel = rel.astype(jnp.int32)
    src = edge_index[0].astype(jnp.int32)
    dst = edge_index[1].astype(jnp.int32)
    full = jnp.full((EP // CH, CH), NP - 1, jnp.int32)
    src4 = lax.dynamic_update_slice(full, src.reshape(E // CH, CH),
                                    (0, 0)).reshape(NS, NGRP, NG, CH)
    dst4 = lax.dynamic_update_slice(full, dst.reshape(E // CH, CH),
                                    (0, 0)).reshape(NS, NGRP, NG, CH)
    dst3d = dst.reshape(NC * NS, DPT, DCH)

    pad = NP - N_ENT
    er = jnp.pad(emb_e_real.astype(f32), ((0, pad), (0, 0)))
    ei = jnp.pad(emb_e_img.astype(f32), ((0, pad), (0, 0)))
    ones_c = jnp.ones((DCH,), f32)
    zeros1 = jnp.zeros((ROWS_PER_TILE,), f32)
    zeros2 = jnp.zeros((ROWS_PER_TILE, D), f32)

    deg_parts, r_r, r_i = _sc_deg_rel(dst3d, rel, emb_rel_real.astype(f32),
                                      emb_rel_img.astype(f32), ones_c, zeros1)
    dvec, h0r, h0i = _tc_prep(deg_parts.T, er, ei)

    s1r, s1i = _sc_scatter(h0r, h0i, src4, dst4, zeros2)
    h1r = _tc_layer(s1r, dvec, W1, do_relu=True, post_scale=True,
                    out_dtype=jnp.float32)
    h1i = _tc_layer(s1i, dvec, W1, do_relu=True, post_scale=True,
                    out_dtype=jnp.float32)

    s2r, s2i = _sc_scatter(h1r, h1i, src4, dst4, zeros2)
    h2r = _tc_layer(s2r, dvec, W2, do_relu=False, post_scale=False,
                    out_dtype=jnp.float32)
    h2i = _tc_layer(s2i, dvec, W2, do_relu=False, post_scale=False,
                    out_dtype=jnp.float32)

    g_r, g_i = _sc_gather_e1(h2r, h2i, e1)
    logits = _tc_score(g_r, g_i, r_r, r_i, er, ei)
    return logits[:, :N_ENT]
